# baseline (speedup 1.0000x reference)
import functools

import jax
import jax.numpy as jnp
from jax import lax
from jax.experimental import pallas as pl
from jax.experimental.pallas import tpu as pltpu
from jax.experimental.pallas import tpu_sc as plsc

T, H, I, E, TOP_K = 2048, 1024, 4096, 8, 2
SOFTCAP = 30.0

B = 128
NBLK = (TOP_K * T) // B + E
P = NBLK * B
IT = 4
TI = I // IT
CHUNK = 256

NC, NS, L = 2, 16, 16
NW = NC * NS
TPW = T // NW
CG = 32
LW = 128

_INV_SQRT2 = 0.7071067811865476


def _gelu_exact(x):
    return x * 0.5 * (1.0 + jax.lax.erf(x * _INV_SQRT2))


def _strict_lower(n, dtype):
    r = lax.broadcasted_iota(jnp.int32, (n, n), 0)
    c = lax.broadcasted_iota(jnp.int32, (n, n), 1)
    return (c < r).astype(dtype)


def _router_body(x_ref, gw_ref, wts_ref, pos_ref, meta_ref):
    x = x_ref[...]
    logits = lax.dot_general(
        x.astype(jnp.bfloat16), gw_ref[...].astype(jnp.bfloat16),
        (((1,), (1,)), ((), ())), preferred_element_type=jnp.float32)
    logits = SOFTCAP * jnp.tanh(logits / SOFTCAP)
    m = jnp.max(logits, axis=-1, keepdims=True)
    ex = jnp.exp(logits - m)
    scores = ex / jnp.sum(ex, axis=-1, keepdims=True)

    eidx = lax.broadcasted_iota(jnp.int32, (T, E), 1)
    a1 = jnp.argmax(scores, axis=-1)
    oh1 = eidx == a1[:, None]
    a2 = jnp.argmax(jnp.where(oh1, -jnp.inf, scores), axis=-1)
    oh2 = eidx == a2[:, None]

    w0 = jnp.sum(jnp.where(oh1, scores, 0.0), axis=-1)
    w1 = jnp.sum(jnp.where(oh2, scores, 0.0), axis=-1)
    wts_ref[...] = jnp.concatenate([w0[:, None], w1[:, None]], axis=1)

    M = (oh1 | oh2).astype(jnp.float32)
    Ls = _strict_lower(CHUNK, jnp.bfloat16)
    cex_chunks = []
    tots = []
    for ci in range(T // CHUNK):
        Mc = lax.slice(M, (ci * CHUNK, 0), ((ci + 1) * CHUNK, E))
        cex_chunks.append(lax.dot_general(
            Ls, Mc.astype(jnp.bfloat16), (((1,), (0,)), ((), ())),
            preferred_element_type=jnp.float32))
        tots.append(jnp.sum(Mc, axis=0, keepdims=True))
    tot = jnp.concatenate(tots, axis=0)
    Lc = _strict_lower(T // CHUNK, jnp.float32)
    base = lax.dot_general(Lc, tot, (((1,), (0,)), ((), ())),
                           preferred_element_type=jnp.float32)
    cexcl = jnp.concatenate(
        [cex_chunks[ci] + lax.slice(base, (ci, 0), (ci + 1, E))
         for ci in range(T // CHUNK)], axis=0)

    counts = jnp.sum(M, axis=0, keepdims=True)
    nb = jnp.ceil(counts * (1.0 / B))
    er = lax.broadcasted_iota(jnp.int32, (E, E), 0)
    ec = lax.broadcasted_iota(jnp.int32, (E, E), 1)
    Le = (er < ec).astype(jnp.float32)
    start_blk = lax.dot_general(nb, Le, (((1,), (0,)), ((), ())),
                                preferred_element_type=jnp.float32)

    posf = start_blk * B + cexcl
    p0 = jnp.sum(jnp.where(oh1, posf, 0.0), axis=-1).astype(jnp.int32)
    p1 = jnp.sum(jnp.where(oh2, posf, 0.0), axis=-1).astype(jnp.int32)
    pos_ref[...] = jnp.concatenate([p0[:, None], p1[:, None]], axis=1)
    meta_ref[...] = jnp.concatenate([start_blk, nb], axis=1).astype(jnp.int32)


def _router(x, gate_w):
    return pl.pallas_call(
        _router_body,
        out_shape=(
            jax.ShapeDtypeStruct((T, TOP_K), jnp.float32),
            jax.ShapeDtypeStruct((T, TOP_K), jnp.int32),
            jax.ShapeDtypeStruct((1, 2 * E), jnp.int32),
        ),
    )(x, gate_w)


def _sc_dispatch_body(xb_hbm, wbig_hbm, pos_hbm, xg_hbm, wsw_hbm,
                      rows_v, wrow_v, idx_v, sem):
    wid = lax.axis_index("s") * NC + lax.axis_index("c")
    base = wid * TPW
    pltpu.sync_copy(xb_hbm.at[pl.ds(base, TPW), :], rows_v)
    pltpu.sync_copy(wbig_hbm.at[wid], wrow_v)
    pltpu.sync_copy(pos_hbm.at[wid], idx_v)
    pltpu.async_copy(rows_v, xg_hbm.at[idx_v.at[0]], sem).wait()
    pltpu.async_copy(rows_v, xg_hbm.at[idx_v.at[1]], sem).wait()
    pltpu.async_copy(wrow_v.at[0], wsw_hbm.at[idx_v.at[0]], sem).wait()
    pltpu.async_copy(wrow_v.at[1], wsw_hbm.at[idx_v.at[1]], sem).wait()


def _sc_dispatch(xb, wbig, pos_sc):
    fn = pl.kernel(
        _sc_dispatch_body,
        out_type=(
            jax.ShapeDtypeStruct((P, H), jnp.float32),
            jax.ShapeDtypeStruct((P, LW), jnp.float32),
        ),
        mesh=plsc.VectorSubcoreMesh(core_axis_name="c", subcore_axis_name="s"),
        scratch_types=[
            pltpu.VMEM((TPW, H), jnp.float32),
            pltpu.VMEM((TOP_K, TPW, LW), jnp.float32),
            pltpu.VMEM((TOP_K, TPW), jnp.int32),
            pltpu.SemaphoreType.DMA,
        ],
    )
    return fn(xb, wbig, pos_sc)


def _ffn_body(meta_ref, xg_ref, wsw_ref, w1_ref, w3_ref, w2_ref, y_ref,
              y_acc, xg_bf, xstage, sem_in, sem_out):
    i = pl.program_id(1)
    e = pl.program_id(0)
    sb = meta_ref[e]
    nb = meta_ref[E + e]
    w1t = w1_ref[0].astype(jnp.bfloat16)
    w3t = w3_ref[0].astype(jnp.bfloat16)
    w2t = w2_ref[0].astype(jnp.bfloat16)

    def ffn_math(rows):
        g = lax.dot_general(rows, w1t, (((1,), (1,)), ((), ())),
                            preferred_element_type=jnp.float32)
        u = lax.dot_general(rows, w3t, (((1,), (1,)), ((), ())),
                            preferred_element_type=jnp.float32)
        act = (_gelu_exact(g) * u).astype(jnp.bfloat16)
        return lax.dot_general(act, w2t, (((1,), (1,)), ((), ())),
                               preferred_element_type=jnp.float32)

    @pl.when(i == 0)
    def _():
        @pl.when(e > 0)
        def _():
            prev_nb = meta_ref[E + e - 1]

            def wt(r, _):
                pltpu.make_async_copy(
                    y_acc.at[pl.ds(r * B, B), :],
                    y_ref.at[pl.ds(r * B, B), :],
                    sem_out).wait()
                return 0
            lax.fori_loop(0, prev_nb, wt, 0)

        @pl.when(nb > 0)
        def _():
            pltpu.make_async_copy(
                xg_ref.at[pl.ds(sb * B, B), :], xstage.at[0], sem_in).start()

        def blk0(r, _):
            pltpu.make_async_copy(
                xg_ref.at[pl.ds((sb + r) * B, B), :],
                xstage.at[r % 2], sem_in).wait()

            @pl.when(r + 1 < nb)
            def _():
                pltpu.make_async_copy(
                    xg_ref.at[pl.ds((sb + r + 1) * B, B), :],
                    xstage.at[(r + 1) % 2], sem_in).start()

            rows = xstage[r % 2].astype(jnp.bfloat16)
            xg_bf[pl.ds(r * B, B), :] = rows
            y_acc[pl.ds(r * B, B), :] = ffn_math(rows)
            return 0
        lax.fori_loop(0, nb, blk0, 0)

    @pl.when(i > 0)
    def _():
        def blk1(r, _):
            rows = xg_bf[pl.ds(r * B, B), :]
            yp = ffn_math(rows)
            val = y_acc[pl.ds(r * B, B), :] + yp
            wcol = wsw_ref[pl.ds((sb + r) * B, B), 0:1]
            val = jnp.where(i == IT - 1, val * wcol, val)
            y_acc[pl.ds(r * B, B), :] = val
            return 0
        lax.fori_loop(0, nb, blk1, 0)

    @pl.when(i == IT - 1)
    def _():
        def cp(r, _):
            pltpu.make_async_copy(
                y_acc.at[pl.ds(r * B, B), :],
                y_ref.at[pl.ds((sb + r) * B, B), :],
                sem_out).start()
            return 0
        lax.fori_loop(0, nb, cp, 0)

        @pl.when(e == E - 1)
        def _():
            def wt(r, _):
                pltpu.make_async_copy(
                    y_acc.at[pl.ds(r * B, B), :],
                    y_ref.at[pl.ds(r * B, B), :],
                    sem_out).wait()
                return 0
            lax.fori_loop(0, nb, wt, 0)


def _ffn(meta_flat, xg, wsw, w1, w3, w2):
    grid_spec = pltpu.PrefetchScalarGridSpec(
        num_scalar_prefetch=1,
        grid=(E, IT),
        in_specs=[
            pl.BlockSpec(memory_space=pltpu.MemorySpace.HBM),
            pl.BlockSpec((P, LW), lambda e, i, meta: (0, 0)),
            pl.BlockSpec((1, TI, H), lambda e, i, meta: (e, i, 0)),
            pl.BlockSpec((1, TI, H), lambda e, i, meta: (e, i, 0)),
            pl.BlockSpec((1, H, TI), lambda e, i, meta: (e, 0, i)),
        ],
        out_specs=pl.BlockSpec(memory_space=pltpu.MemorySpace.HBM),
        scratch_shapes=[
            pltpu.VMEM((T, H), jnp.float32),
            pltpu.VMEM((T, H), jnp.bfloat16),
            pltpu.VMEM((2, B, H), jnp.float32),
            pltpu.SemaphoreType.DMA,
            pltpu.SemaphoreType.DMA,
        ],
    )
    return pl.pallas_call(
        _ffn_body,
        grid_spec=grid_spec,
        out_shape=jax.ShapeDtypeStruct((P, H), jnp.float32),
    )(meta_flat, xg, wsw, w1, w3, w2)


def _sc_combine_body(y_hbm, pos_hbm, out_hbm, idx_v, rows0_v, rows1_v,
                     out_v, sem0, sem1):
    wid = lax.axis_index("s") * NC + lax.axis_index("c")
    base = wid * TPW
    pltpu.sync_copy(pos_hbm.at[wid], idx_v)
    for c in range(TPW // CG):
        cp0 = pltpu.async_copy(
            y_hbm.at[idx_v.at[0, pl.ds(c * CG, CG)]], rows0_v, sem0)
        cp1 = pltpu.async_copy(
            y_hbm.at[idx_v.at[1, pl.ds(c * CG, CG)]], rows1_v, sem1)
        cp0.wait()
        cp1.wait()

        def lane(l, _):
            for j in range(CG):
                sl = pl.ds(l * L, L)
                out_v[j, sl] = rows0_v[j, sl] + rows1_v[j, sl]
            return 0

        lax.fori_loop(0, H // L, lane, 0)
        pltpu.sync_copy(out_v, out_hbm.at[pl.ds(base + c * CG, CG), :])


def _sc_combine(y, pos_sc):
    fn = pl.kernel(
        _sc_combine_body,
        out_type=jax.ShapeDtypeStruct((T, H), jnp.float32),
        mesh=plsc.VectorSubcoreMesh(core_axis_name="c", subcore_axis_name="s"),
        scratch_types=[
            pltpu.VMEM((TOP_K, TPW), jnp.int32),
            pltpu.VMEM((CG, H), jnp.float32),
            pltpu.VMEM((CG, H), jnp.float32),
            pltpu.VMEM((CG, H), jnp.float32),
            pltpu.SemaphoreType.DMA,
            pltpu.SemaphoreType.DMA,
        ],
    )
    return fn(y, pos_sc)


def kernel(hidden_states, gate_w, w1, w3, w2):
    wts, pos, meta = _router(hidden_states, gate_w)
    pos_sc = pos.T.reshape(TOP_K, NW, TPW).transpose(1, 0, 2)
    wbig = jnp.broadcast_to(
        wts.T.reshape(TOP_K, NW, TPW).transpose(1, 0, 2)[..., None],
        (NW, TOP_K, TPW, LW))
    xg, wsw = _sc_dispatch(hidden_states, wbig, pos_sc)
    y = _ffn(meta.reshape(2 * E), xg, wsw, w1, w3, w2)
    out = _sc_combine(y, pos_sc)
    return out


# speedup vs baseline: 1.5276x; 1.5276x over previous
import functools

import jax
import jax.numpy as jnp
from jax import lax
from jax.experimental import pallas as pl
from jax.experimental.pallas import tpu as pltpu
from jax.experimental.pallas import tpu_sc as plsc

T, H, I, E, TOP_K = 2048, 1024, 4096, 8, 2
SOFTCAP = 30.0

B = 512
NBLK = (TOP_K * T) // B + E
P = NBLK * B
IT = 4
TI = I // IT
CHUNK = 256

NC, NS, L = 2, 16, 16
NW = NC * NS
TPW = T // NW
CG = 32
LW = 128

_INV_SQRT2 = 0.7071067811865476


def _gelu_exact(x):
    return x * 0.5 * (1.0 + jax.lax.erf(x * _INV_SQRT2))


def _strict_lower(n, dtype):
    r = lax.broadcasted_iota(jnp.int32, (n, n), 0)
    c = lax.broadcasted_iota(jnp.int32, (n, n), 1)
    return (c < r).astype(dtype)


def _router_body(x_ref, gw_ref, wts_ref, pos_ref, meta_ref):
    x = x_ref[...]
    logits = lax.dot_general(
        x.astype(jnp.bfloat16), gw_ref[...].astype(jnp.bfloat16),
        (((1,), (1,)), ((), ())), preferred_element_type=jnp.float32)
    logits = SOFTCAP * jnp.tanh(logits / SOFTCAP)
    m = jnp.max(logits, axis=-1, keepdims=True)
    ex = jnp.exp(logits - m)
    scores = ex / jnp.sum(ex, axis=-1, keepdims=True)

    eidx = lax.broadcasted_iota(jnp.int32, (T, E), 1)
    a1 = jnp.argmax(scores, axis=-1)
    oh1 = eidx == a1[:, None]
    a2 = jnp.argmax(jnp.where(oh1, -jnp.inf, scores), axis=-1)
    oh2 = eidx == a2[:, None]

    w0 = jnp.sum(jnp.where(oh1, scores, 0.0), axis=-1)
    w1 = jnp.sum(jnp.where(oh2, scores, 0.0), axis=-1)
    wts_ref[...] = jnp.concatenate([w0[:, None], w1[:, None]], axis=1)

    M = (oh1 | oh2).astype(jnp.float32)
    Ls = _strict_lower(CHUNK, jnp.bfloat16)
    cex_chunks = []
    tots = []
    for ci in range(T // CHUNK):
        Mc = lax.slice(M, (ci * CHUNK, 0), ((ci + 1) * CHUNK, E))
        cex_chunks.append(lax.dot_general(
            Ls, Mc.astype(jnp.bfloat16), (((1,), (0,)), ((), ())),
            preferred_element_type=jnp.float32))
        tots.append(jnp.sum(Mc, axis=0, keepdims=True))
    tot = jnp.concatenate(tots, axis=0)
    Lc = _strict_lower(T // CHUNK, jnp.float32)
    base = lax.dot_general(Lc, tot, (((1,), (0,)), ((), ())),
                           preferred_element_type=jnp.float32)
    cexcl = jnp.concatenate(
        [cex_chunks[ci] + lax.slice(base, (ci, 0), (ci + 1, E))
         for ci in range(T // CHUNK)], axis=0)

    counts = jnp.sum(M, axis=0, keepdims=True)
    nb = jnp.ceil(counts * (1.0 / B))
    er = lax.broadcasted_iota(jnp.int32, (E, E), 0)
    ec = lax.broadcasted_iota(jnp.int32, (E, E), 1)
    Le = (er < ec).astype(jnp.float32)
    start_blk = lax.dot_general(nb, Le, (((1,), (0,)), ((), ())),
                                preferred_element_type=jnp.float32)

    posf = start_blk * B + cexcl
    p0 = jnp.sum(jnp.where(oh1, posf, 0.0), axis=-1).astype(jnp.int32)
    p1 = jnp.sum(jnp.where(oh2, posf, 0.0), axis=-1).astype(jnp.int32)
    pos_ref[...] = jnp.concatenate([p0[:, None], p1[:, None]], axis=1)
    meta_ref[...] = jnp.concatenate([start_blk, nb], axis=1).astype(jnp.int32)


def _router(x, gate_w):
    return pl.pallas_call(
        _router_body,
        out_shape=(
            jax.ShapeDtypeStruct((T, TOP_K), jnp.float32),
            jax.ShapeDtypeStruct((T, TOP_K), jnp.int32),
            jax.ShapeDtypeStruct((1, 2 * E), jnp.int32),
        ),
    )(x, gate_w)


def _sc_dispatch_body(xb_hbm, wbig_hbm, pos_hbm, xg_hbm, wsw_hbm,
                      rows_v, wrow_v, idx_v, sem):
    wid = lax.axis_index("s") * NC + lax.axis_index("c")
    base = wid * TPW
    pltpu.sync_copy(xb_hbm.at[pl.ds(base, TPW), :], rows_v)
    pltpu.sync_copy(wbig_hbm.at[wid], wrow_v)
    pltpu.sync_copy(pos_hbm.at[wid], idx_v)
    pltpu.async_copy(rows_v, xg_hbm.at[idx_v.at[0]], sem).wait()
    pltpu.async_copy(rows_v, xg_hbm.at[idx_v.at[1]], sem).wait()
    pltpu.async_copy(wrow_v.at[0], wsw_hbm.at[idx_v.at[0]], sem).wait()
    pltpu.async_copy(wrow_v.at[1], wsw_hbm.at[idx_v.at[1]], sem).wait()


def _sc_dispatch(xb, wbig, pos_sc):
    fn = pl.kernel(
        _sc_dispatch_body,
        out_type=(
            jax.ShapeDtypeStruct((P, H), jnp.float32),
            jax.ShapeDtypeStruct((P, LW), jnp.float32),
        ),
        mesh=plsc.VectorSubcoreMesh(core_axis_name="c", subcore_axis_name="s"),
        scratch_types=[
            pltpu.VMEM((TPW, H), jnp.float32),
            pltpu.VMEM((TOP_K, TPW, LW), jnp.float32),
            pltpu.VMEM((TOP_K, TPW), jnp.int32),
            pltpu.SemaphoreType.DMA,
        ],
    )
    return fn(xb, wbig, pos_sc)


def _ffn_body(meta_ref, xg_ref, wsw_ref, w1_ref, w3_ref, w2_ref, y_ref,
              y_acc, xg_bf, xstage, sem_in, sem_out):
    i = pl.program_id(1)
    e = pl.program_id(0)
    sb = meta_ref[e]
    nb = meta_ref[E + e]
    w1t = w1_ref[0].astype(jnp.bfloat16)
    w3t = w3_ref[0].astype(jnp.bfloat16)
    w2t = w2_ref[0].astype(jnp.bfloat16)

    def ffn_math(rows):
        g = lax.dot_general(rows, w1t, (((1,), (1,)), ((), ())),
                            preferred_element_type=jnp.float32)
        u = lax.dot_general(rows, w3t, (((1,), (1,)), ((), ())),
                            preferred_element_type=jnp.float32)
        act = (_gelu_exact(g) * u).astype(jnp.bfloat16)
        return lax.dot_general(act, w2t, (((1,), (1,)), ((), ())),
                               preferred_element_type=jnp.float32)

    @pl.when(i == 0)
    def _():
        @pl.when(e > 0)
        def _():
            prev_nb = meta_ref[E + e - 1]

            def wt(r, _):
                pltpu.make_async_copy(
                    y_acc.at[pl.ds(r * B, B), :],
                    y_ref.at[pl.ds(r * B, B), :],
                    sem_out).wait()
                return 0
            lax.fori_loop(0, prev_nb, wt, 0)

        @pl.when(nb > 0)
        def _():
            pltpu.make_async_copy(
                xg_ref.at[pl.ds(sb * B, B), :], xstage.at[0], sem_in).start()

        def blk0(r, _):
            pltpu.make_async_copy(
                xg_ref.at[pl.ds((sb + r) * B, B), :],
                xstage.at[r % 2], sem_in).wait()

            @pl.when(r + 1 < nb)
            def _():
                pltpu.make_async_copy(
                    xg_ref.at[pl.ds((sb + r + 1) * B, B), :],
                    xstage.at[(r + 1) % 2], sem_in).start()

            rows = xstage[r % 2].astype(jnp.bfloat16)
            xg_bf[pl.ds(r * B, B), :] = rows
            y_acc[pl.ds(r * B, B), :] = ffn_math(rows)
            return 0
        lax.fori_loop(0, nb, blk0, 0)

    @pl.when(i > 0)
    def _():
        def blk1(r, _):
            rows = xg_bf[pl.ds(r * B, B), :]
            yp = ffn_math(rows)
            val = y_acc[pl.ds(r * B, B), :] + yp
            wcol = wsw_ref[pl.ds((sb + r) * B, B), 0:1]
            val = jnp.where(i == IT - 1, val * wcol, val)
            y_acc[pl.ds(r * B, B), :] = val
            return 0
        lax.fori_loop(0, nb, blk1, 0)

    @pl.when(i == IT - 1)
    def _():
        def cp(r, _):
            pltpu.make_async_copy(
                y_acc.at[pl.ds(r * B, B), :],
                y_ref.at[pl.ds((sb + r) * B, B), :],
                sem_out).start()
            return 0
        lax.fori_loop(0, nb, cp, 0)

        @pl.when(e == E - 1)
        def _():
            def wt(r, _):
                pltpu.make_async_copy(
                    y_acc.at[pl.ds(r * B, B), :],
                    y_ref.at[pl.ds(r * B, B), :],
                    sem_out).wait()
                return 0
            lax.fori_loop(0, nb, wt, 0)


def _ffn(meta_flat, xg, wsw, w1, w3, w2):
    grid_spec = pltpu.PrefetchScalarGridSpec(
        num_scalar_prefetch=1,
        grid=(E, IT),
        in_specs=[
            pl.BlockSpec(memory_space=pltpu.MemorySpace.HBM),
            pl.BlockSpec((P, LW), lambda e, i, meta: (0, 0)),
            pl.BlockSpec((1, TI, H), lambda e, i, meta: (e, i, 0)),
            pl.BlockSpec((1, TI, H), lambda e, i, meta: (e, i, 0)),
            pl.BlockSpec((1, H, TI), lambda e, i, meta: (e, 0, i)),
        ],
        out_specs=pl.BlockSpec(memory_space=pltpu.MemorySpace.HBM),
        scratch_shapes=[
            pltpu.VMEM((T, H), jnp.float32),
            pltpu.VMEM((T, H), jnp.bfloat16),
            pltpu.VMEM((2, B, H), jnp.float32),
            pltpu.SemaphoreType.DMA,
            pltpu.SemaphoreType.DMA,
        ],
    )
    return pl.pallas_call(
        _ffn_body,
        grid_spec=grid_spec,
        out_shape=jax.ShapeDtypeStruct((P, H), jnp.float32),
    )(meta_flat, xg, wsw, w1, w3, w2)


def _sc_combine_body(y_hbm, pos_hbm, out_hbm, idx_v, rows0_v, rows1_v,
                     out_v, sem0, sem1):
    wid = lax.axis_index("s") * NC + lax.axis_index("c")
    base = wid * TPW
    pltpu.sync_copy(pos_hbm.at[wid], idx_v)
    for c in range(TPW // CG):
        cp0 = pltpu.async_copy(
            y_hbm.at[idx_v.at[0, pl.ds(c * CG, CG)]], rows0_v, sem0)
        cp1 = pltpu.async_copy(
            y_hbm.at[idx_v.at[1, pl.ds(c * CG, CG)]], rows1_v, sem1)
        cp0.wait()
        cp1.wait()

        def lane(l, _):
            for j in range(CG):
                sl = pl.ds(l * L, L)
                out_v[j, sl] = rows0_v[j, sl] + rows1_v[j, sl]
            return 0

        lax.fori_loop(0, H // L, lane, 0)
        pltpu.sync_copy(out_v, out_hbm.at[pl.ds(base + c * CG, CG), :])


def _sc_combine(y, pos_sc):
    fn = pl.kernel(
        _sc_combine_body,
        out_type=jax.ShapeDtypeStruct((T, H), jnp.float32),
        mesh=plsc.VectorSubcoreMesh(core_axis_name="c", subcore_axis_name="s"),
        scratch_types=[
            pltpu.VMEM((TOP_K, TPW), jnp.int32),
            pltpu.VMEM((CG, H), jnp.float32),
            pltpu.VMEM((CG, H), jnp.float32),
            pltpu.VMEM((CG, H), jnp.float32),
            pltpu.SemaphoreType.DMA,
            pltpu.SemaphoreType.DMA,
        ],
    )
    return fn(y, pos_sc)


def kernel(hidden_states, gate_w, w1, w3, w2):
    wts, pos, meta = _router(hidden_states, gate_w)
    pos_sc = pos.T.reshape(TOP_K, NW, TPW).transpose(1, 0, 2)
    wbig = jnp.broadcast_to(
        wts.T.reshape(TOP_K, NW, TPW).transpose(1, 0, 2)[..., None],
        (NW, TOP_K, TPW, LW))
    xg, wsw = _sc_dispatch(hidden_states, wbig, pos_sc)
    y = _ffn(meta.reshape(2 * E), xg, wsw, w1, w3, w2)
    out = _sc_combine(y, pos_sc)
    return out


# speedup vs baseline: 1.6404x; 1.0738x over previous
import functools

import jax
import jax.numpy as jnp
from jax import lax
from jax.experimental import pallas as pl
from jax.experimental.pallas import tpu as pltpu
from jax.experimental.pallas import tpu_sc as plsc

T, H, I, E, TOP_K = 2048, 1024, 4096, 8, 2
SOFTCAP = 30.0

B = 256
NBLK = (TOP_K * T) // B + E
P = NBLK * B
IT = 4
TI = I // IT
CHUNK = 256

NC, NS, L = 2, 16, 16
NW = NC * NS
TPW = T // NW
CG = 16
LW = 128

_INV_SQRT2 = 0.7071067811865476


def _gelu_exact(x):
    return x * 0.5 * (1.0 + jax.lax.erf(x * _INV_SQRT2))


def _strict_lower(n, dtype):
    r = lax.broadcasted_iota(jnp.int32, (n, n), 0)
    c = lax.broadcasted_iota(jnp.int32, (n, n), 1)
    return (c < r).astype(dtype)


def _router_body(x_ref, gw_ref, wts_ref, pos_ref, meta_ref):
    x = x_ref[...]
    logits = lax.dot_general(
        x.astype(jnp.bfloat16), gw_ref[...].astype(jnp.bfloat16),
        (((1,), (1,)), ((), ())), preferred_element_type=jnp.float32)
    logits = SOFTCAP * jnp.tanh(logits / SOFTCAP)
    m = jnp.max(logits, axis=-1, keepdims=True)
    ex = jnp.exp(logits - m)
    scores = ex / jnp.sum(ex, axis=-1, keepdims=True)

    eidx = lax.broadcasted_iota(jnp.int32, (T, E), 1)
    a1 = jnp.argmax(scores, axis=-1)
    oh1 = eidx == a1[:, None]
    a2 = jnp.argmax(jnp.where(oh1, -jnp.inf, scores), axis=-1)
    oh2 = eidx == a2[:, None]

    w0 = jnp.sum(jnp.where(oh1, scores, 0.0), axis=-1)
    w1 = jnp.sum(jnp.where(oh2, scores, 0.0), axis=-1)
    wts_ref[...] = jnp.concatenate([w0[:, None], w1[:, None]], axis=1)

    M = (oh1 | oh2).astype(jnp.float32)
    Ls = _strict_lower(CHUNK, jnp.bfloat16)
    cex_chunks = []
    tots = []
    for ci in range(T // CHUNK):
        Mc = lax.slice(M, (ci * CHUNK, 0), ((ci + 1) * CHUNK, E))
        cex_chunks.append(lax.dot_general(
            Ls, Mc.astype(jnp.bfloat16), (((1,), (0,)), ((), ())),
            preferred_element_type=jnp.float32))
        tots.append(jnp.sum(Mc, axis=0, keepdims=True))
    tot = jnp.concatenate(tots, axis=0)
    Lc = _strict_lower(T // CHUNK, jnp.float32)
    base = lax.dot_general(Lc, tot, (((1,), (0,)), ((), ())),
                           preferred_element_type=jnp.float32)
    cexcl = jnp.concatenate(
        [cex_chunks[ci] + lax.slice(base, (ci, 0), (ci + 1, E))
         for ci in range(T // CHUNK)], axis=0)

    counts = jnp.sum(M, axis=0, keepdims=True)
    nb = jnp.ceil(counts * (1.0 / B))
    er = lax.broadcasted_iota(jnp.int32, (E, E), 0)
    ec = lax.broadcasted_iota(jnp.int32, (E, E), 1)
    Le = (er < ec).astype(jnp.float32)
    start_blk = lax.dot_general(nb, Le, (((1,), (0,)), ((), ())),
                                preferred_element_type=jnp.float32)

    posf = start_blk * B + cexcl
    p0 = jnp.sum(jnp.where(oh1, posf, 0.0), axis=-1).astype(jnp.int32)
    p1 = jnp.sum(jnp.where(oh2, posf, 0.0), axis=-1).astype(jnp.int32)
    pos_ref[...] = jnp.concatenate([p0[:, None], p1[:, None]], axis=1)
    meta_ref[...] = jnp.concatenate([start_blk, nb], axis=1).astype(jnp.int32)


def _router(x, gate_w):
    return pl.pallas_call(
        _router_body,
        out_shape=(
            jax.ShapeDtypeStruct((T, TOP_K), jnp.float32),
            jax.ShapeDtypeStruct((T, TOP_K), jnp.int32),
            jax.ShapeDtypeStruct((1, 2 * E), jnp.int32),
        ),
    )(x, gate_w)


def _sc_dispatch_body(xb_hbm, wbig_hbm, pos_hbm, xg_hbm, wsw_hbm,
                      rows_v, wrow_v, idx_v, sem):
    wid = lax.axis_index("s") * NC + lax.axis_index("c")
    base = wid * TPW
    pltpu.sync_copy(xb_hbm.at[pl.ds(base, TPW), :], rows_v)
    pltpu.sync_copy(wbig_hbm.at[wid], wrow_v)
    pltpu.sync_copy(pos_hbm.at[wid], idx_v)
    cps = [
        pltpu.async_copy(rows_v, xg_hbm.at[idx_v.at[0]], sem),
        pltpu.async_copy(rows_v, xg_hbm.at[idx_v.at[1]], sem),
        pltpu.async_copy(wrow_v.at[0], wsw_hbm.at[idx_v.at[0]], sem),
        pltpu.async_copy(wrow_v.at[1], wsw_hbm.at[idx_v.at[1]], sem),
    ]
    for cp in cps:
        cp.wait()


def _sc_dispatch(xb, wbig, pos_sc):
    fn = pl.kernel(
        _sc_dispatch_body,
        out_type=(
            jax.ShapeDtypeStruct((P, H), jnp.float32),
            jax.ShapeDtypeStruct((P, LW), jnp.float32),
        ),
        mesh=plsc.VectorSubcoreMesh(core_axis_name="c", subcore_axis_name="s"),
        scratch_types=[
            pltpu.VMEM((TPW, H), jnp.float32),
            pltpu.VMEM((TOP_K, TPW, LW), jnp.float32),
            pltpu.VMEM((TOP_K, TPW), jnp.int32),
            pltpu.SemaphoreType.DMA,
        ],
    )
    return fn(xb, wbig, pos_sc)


def _ffn_body(meta_ref, xg_ref, wsw_ref, w1_ref, w3_ref, w2_ref, y_ref,
              y_acc, xg_bf, xstage, sem_in, sem_out):
    i = pl.program_id(1)
    e = pl.program_id(0)
    sb = meta_ref[e]
    nb = meta_ref[E + e]
    w1t = w1_ref[0].astype(jnp.bfloat16)
    w3t = w3_ref[0].astype(jnp.bfloat16)
    w2t = w2_ref[0].astype(jnp.bfloat16)

    def ffn_math(rows):
        g = lax.dot_general(rows, w1t, (((1,), (1,)), ((), ())),
                            preferred_element_type=jnp.float32)
        u = lax.dot_general(rows, w3t, (((1,), (1,)), ((), ())),
                            preferred_element_type=jnp.float32)
        act = (_gelu_exact(g) * u).astype(jnp.bfloat16)
        return lax.dot_general(act, w2t, (((1,), (1,)), ((), ())),
                               preferred_element_type=jnp.float32)

    @pl.when(i == 0)
    def _():
        @pl.when(e > 0)
        def _():
            prev_nb = meta_ref[E + e - 1]

            def wt(r, _):
                pltpu.make_async_copy(
                    y_acc.at[pl.ds(r * B, B), :],
                    y_ref.at[pl.ds(r * B, B), :],
                    sem_out).wait()
                return 0
            lax.fori_loop(0, prev_nb, wt, 0)

        @pl.when(nb > 0)
        def _():
            pltpu.make_async_copy(
                xg_ref.at[pl.ds(sb * B, B), :], xstage.at[0], sem_in).start()

        def blk0(r, _):
            pltpu.make_async_copy(
                xg_ref.at[pl.ds((sb + r) * B, B), :],
                xstage.at[r % 2], sem_in).wait()

            @pl.when(r + 1 < nb)
            def _():
                pltpu.make_async_copy(
                    xg_ref.at[pl.ds((sb + r + 1) * B, B), :],
                    xstage.at[(r + 1) % 2], sem_in).start()

            rows = xstage[r % 2].astype(jnp.bfloat16)
            xg_bf[pl.ds(r * B, B), :] = rows
            y_acc[pl.ds(r * B, B), :] = ffn_math(rows)
            return 0
        lax.fori_loop(0, nb, blk0, 0)

    @pl.when(i > 0)
    def _():
        def blk1(r, _):
            rows = xg_bf[pl.ds(r * B, B), :]
            yp = ffn_math(rows)
            val = y_acc[pl.ds(r * B, B), :] + yp
            wcol = wsw_ref[pl.ds((sb + r) * B, B), 0:1]
            val = jnp.where(i == IT - 1, val * wcol, val)
            y_acc[pl.ds(r * B, B), :] = val
            return 0
        lax.fori_loop(0, nb, blk1, 0)

    @pl.when(i == IT - 1)
    def _():
        def cp(r, _):
            pltpu.make_async_copy(
                y_acc.at[pl.ds(r * B, B), :],
                y_ref.at[pl.ds((sb + r) * B, B), :],
                sem_out).start()
            return 0
        lax.fori_loop(0, nb, cp, 0)

        @pl.when(e == E - 1)
        def _():
            def wt(r, _):
                pltpu.make_async_copy(
                    y_acc.at[pl.ds(r * B, B), :],
                    y_ref.at[pl.ds(r * B, B), :],
                    sem_out).wait()
                return 0
            lax.fori_loop(0, nb, wt, 0)


def _ffn(meta_flat, xg, wsw, w1, w3, w2):
    grid_spec = pltpu.PrefetchScalarGridSpec(
        num_scalar_prefetch=1,
        grid=(E, IT),
        in_specs=[
            pl.BlockSpec(memory_space=pltpu.MemorySpace.HBM),
            pl.BlockSpec((P, LW), lambda e, i, meta: (0, 0)),
            pl.BlockSpec((1, TI, H), lambda e, i, meta: (e, i, 0)),
            pl.BlockSpec((1, TI, H), lambda e, i, meta: (e, i, 0)),
            pl.BlockSpec((1, H, TI), lambda e, i, meta: (e, 0, i)),
        ],
        out_specs=pl.BlockSpec(memory_space=pltpu.MemorySpace.HBM),
        scratch_shapes=[
            pltpu.VMEM((T, H), jnp.float32),
            pltpu.VMEM((T, H), jnp.bfloat16),
            pltpu.VMEM((2, B, H), jnp.float32),
            pltpu.SemaphoreType.DMA,
            pltpu.SemaphoreType.DMA,
        ],
    )
    return pl.pallas_call(
        _ffn_body,
        grid_spec=grid_spec,
        out_shape=jax.ShapeDtypeStruct((P, H), jnp.float32),
    )(meta_flat, xg, wsw, w1, w3, w2)


def _sc_combine_body(y_hbm, pos_hbm, out_hbm, idx_v, rows_v, out_v,
                     sem0, sem1):
    wid = lax.axis_index("s") * NC + lax.axis_index("c")
    base = wid * TPW
    pltpu.sync_copy(pos_hbm.at[wid], idx_v)
    nchunk = TPW // CG

    def gathers(c, buf):
        return (
            pltpu.async_copy(
                y_hbm.at[idx_v.at[0, pl.ds(c * CG, CG)]], rows_v.at[buf, 0],
                sem0),
            pltpu.async_copy(
                y_hbm.at[idx_v.at[1, pl.ds(c * CG, CG)]], rows_v.at[buf, 1],
                sem1),
        )

    gathers(0, 0)
    for c in range(nchunk):
        b = c % 2
        pltpu.make_async_copy(
            y_hbm.at[idx_v.at[0, pl.ds(c * CG, CG)]], rows_v.at[b, 0],
            sem0).wait()
        pltpu.make_async_copy(
            y_hbm.at[idx_v.at[1, pl.ds(c * CG, CG)]], rows_v.at[b, 1],
            sem1).wait()
        if c + 1 < nchunk:
            gathers(c + 1, (c + 1) % 2)

        def lane(l, _):
            for j in range(CG):
                sl = pl.ds(l * L, L)
                out_v[j, sl] = rows_v[b, 0, j, sl] + rows_v[b, 1, j, sl]
            return 0

        lax.fori_loop(0, H // L, lane, 0)
        pltpu.sync_copy(out_v, out_hbm.at[pl.ds(base + c * CG, CG), :])


def _sc_combine(y, pos_sc):
    fn = pl.kernel(
        _sc_combine_body,
        out_type=jax.ShapeDtypeStruct((T, H), jnp.float32),
        mesh=plsc.VectorSubcoreMesh(core_axis_name="c", subcore_axis_name="s"),
        scratch_types=[
            pltpu.VMEM((TOP_K, TPW), jnp.int32),
            pltpu.VMEM((2, TOP_K, CG, H), jnp.float32),
            pltpu.VMEM((CG, H), jnp.float32),
            pltpu.SemaphoreType.DMA,
            pltpu.SemaphoreType.DMA,
        ],
    )
    return fn(y, pos_sc)


def kernel(hidden_states, gate_w, w1, w3, w2):
    wts, pos, meta = _router(hidden_states, gate_w)
    pos_sc = pos.T.reshape(TOP_K, NW, TPW).transpose(1, 0, 2)
    wbig = jnp.broadcast_to(
        wts.T.reshape(TOP_K, NW, TPW).transpose(1, 0, 2)[..., None],
        (NW, TOP_K, TPW, LW))
    xg, wsw = _sc_dispatch(hidden_states, wbig, pos_sc)
    y = _ffn(meta.reshape(2 * E), xg, wsw, w1, w3, w2)
    out = _sc_combine(y, pos_sc)
    return out
